# baseline (speedup 1.0000x reference)
import functools

import jax
import jax.numpy as jnp
from jax.experimental import pallas as pl
from jax.experimental.pallas import tpu as pltpu


def _ssm_body(x_ref, apow_ref, bt_ref, ct_ref, o_ref, uh_ref, hcarry_ref,
              *, grp, ct, k, log2k, nsup):
    f32 = jnp.float32
    bf16 = jnp.bfloat16
    s = uh_ref.shape[-1]
    rows = grp * ct

    @pl.when(pl.program_id(1) == 0)
    def _init():
        hcarry_ref[...] = jnp.zeros_like(hcarry_ref)

    xv = x_ref[...].reshape(rows, x_ref.shape[-1]).astype(bf16)
    uh_ref[...] = jnp.dot(xv, bt_ref[...],
                          preferred_element_type=f32).reshape(grp, ct, s)

    uh_ref[:, 0:1, :] = uh_ref[:, 0:1, :] + hcarry_ref[...][:, None, :]

    for j in range(1, log2k + 1):
        m = 1 << (j - 1)
        if m % 8 == 0:
            srows = grp * (ct - m)
            w = jnp.dot(uh_ref[:, :ct - m, :].reshape(srows, s).astype(bf16),
                        apow_ref[j - 1],
                        preferred_element_type=f32).reshape(grp, ct - m, s)
            uh_ref[:, m:, :] = uh_ref[:, m:, :] + w
        else:
            w = jnp.dot(uh_ref[...].reshape(rows, s).astype(bf16),
                        apow_ref[j - 1],
                        preferred_element_type=f32).reshape(grp, ct, s)
            uh_ref[:, m:, :] = uh_ref[:, m:, :] + w[:, :ct - m, :]

    h = uh_ref[:, 0:k, :].reshape(grp * k, s)
    for j in range(nsup):
        if j > 0:
            h = (jnp.dot(h.astype(bf16), apow_ref[log2k],
                         preferred_element_type=f32)
                 + uh_ref[:, j * k:(j + 1) * k, :].reshape(grp * k, s))
        o_ref[:, j * k:(j + 1) * k, :] = jnp.dot(
            h.astype(bf16), ct_ref[...],
            preferred_element_type=f32).astype(
                o_ref.dtype).reshape(grp, k, o_ref.shape[-1])

    hcarry_ref[...] = jnp.dot(
        h.reshape(grp, k, s)[:, k - 1, :].astype(bf16), apow_ref[0],
        preferred_element_type=f32)


def kernel(x, A, B, C):
    bsz, T, input_dim = x.shape
    state_dim = A.shape[0]
    out_dim = C.shape[0]
    out_dtype = x.dtype
    f32 = jnp.float32
    bf16 = jnp.bfloat16

    K = 32
    LOG2K = 5
    CT = 256

    G = 2 if bsz >= 16 else 1
    grp = ((-(-bsz // G) + 7) // 8) * 8
    bsz_pad = grp * G
    T_pad = -(-T // CT) * CT
    num_chunks = T_pad // CT
    nsup = CT // K
    out_pad = ((out_dim + 127) // 128) * 128

    xp = x
    if bsz_pad != bsz or T_pad != T:
        xp = jnp.pad(x, ((0, bsz_pad - bsz), (0, T_pad - T), (0, 0)))
    x_g = xp.reshape(G, grp, T_pad, input_dim)

    apows = [jnp.transpose(A).astype(f32)]
    for _ in range(LOG2K):
        apows.append(apows[-1] @ apows[-1])
    apow_t = jnp.stack(apows, axis=0).astype(bf16)
    b_t = jnp.transpose(B).astype(bf16)
    c_t = jnp.pad(jnp.transpose(C),
                  ((0, 0), (0, out_pad - out_dim))).astype(bf16)

    body = functools.partial(_ssm_body, grp=grp, ct=CT, k=K, log2k=LOG2K,
                             nsup=nsup)

    est_vmem = (2 * grp * CT * (input_dim * 4 + out_pad * 4)
                + grp * CT * state_dim * 4
                + 2 * (apow_t.size + b_t.size + c_t.size) * 2)
    vmem_limit = int(min(max(2 * est_vmem, 32 << 20), 64 << 20))

    y = pl.pallas_call(
        body,
        out_shape=jax.ShapeDtypeStruct((G, grp, T_pad, out_pad), out_dtype),
        grid_spec=pltpu.PrefetchScalarGridSpec(
            num_scalar_prefetch=0,
            grid=(G, num_chunks),
            in_specs=[
                pl.BlockSpec((None, grp, CT, input_dim),
                             lambda g, c: (g, 0, c, 0)),
                pl.BlockSpec((LOG2K + 1, state_dim, state_dim),
                             lambda g, c: (0, 0, 0)),
                pl.BlockSpec((input_dim, state_dim), lambda g, c: (0, 0)),
                pl.BlockSpec((state_dim, out_pad), lambda g, c: (0, 0)),
            ],
            out_specs=pl.BlockSpec((None, grp, CT, out_pad),
                                   lambda g, c: (g, 0, c, 0)),
            scratch_shapes=[
                pltpu.VMEM((grp, CT, state_dim), f32),
                pltpu.VMEM((grp, state_dim), f32),
            ],
        ),
        compiler_params=pltpu.CompilerParams(
            dimension_semantics=("parallel", "arbitrary"),
            vmem_limit_bytes=vmem_limit,
            flags={"XLA_TPU_STORE_TO_LOAD_FORWARDING_WINDOW": 12288},
        ),
    )(x_g, apow_t, b_t, c_t)

    y = y.reshape(bsz_pad, T_pad, out_pad)[:bsz, :T, :out_dim]
    return y


# speedup vs baseline: 1.0103x; 1.0103x over previous
import functools

import jax
import jax.numpy as jnp
from jax.experimental import pallas as pl
from jax.experimental.pallas import tpu as pltpu


def _ssm_body(x_ref, btaug_ref, apow_ref, acat_ref, ct_ref, o_ref,
              uh_ref, hcarry_ref, *, grp, ct, k, log2k, nsup):
    f32 = jnp.float32
    bf16 = jnp.bfloat16
    s = uh_ref.shape[-1]
    rows = grp * ct

    @pl.when(pl.program_id(1) == 0)
    def _init():
        hcarry_ref[...] = jnp.zeros_like(hcarry_ref)

    xv = x_ref[...].reshape(rows, x_ref.shape[-1]).astype(bf16)
    uau = jnp.dot(xv, btaug_ref[...], preferred_element_type=f32)
    u3 = uau[:, :s].reshape(grp, ct, s)
    w3 = uau[:, s:].reshape(grp, ct, s)
    uh_ref[...] = u3
    uh_ref[:, 1:, :] = uh_ref[:, 1:, :] + w3[:, :ct - 1, :]
    cc = hcarry_ref[...]
    uh_ref[:, 0:1, :] = uh_ref[:, 0:1, :] + cc[:, :s][:, None, :]
    uh_ref[:, 1:2, :] = uh_ref[:, 1:2, :] + cc[:, s:][:, None, :]

    for j in range(2, log2k + 1):
        m = 1 << (j - 1)
        if m % 8 == 0:
            srows = grp * (ct - m)
            w = jnp.dot(uh_ref[:, :ct - m, :].reshape(srows, s).astype(bf16),
                        apow_ref[j - 2],
                        preferred_element_type=f32).reshape(grp, ct - m, s)
            uh_ref[:, m:, :] = uh_ref[:, m:, :] + w
        else:
            w = jnp.dot(uh_ref[...].reshape(rows, s).astype(bf16),
                        apow_ref[j - 2],
                        preferred_element_type=f32).reshape(grp, ct, s)
            uh_ref[:, m:, :] = uh_ref[:, m:, :] + w[:, :ct - m, :]

    h = uh_ref[:, 0:k, :].reshape(grp * k, s)
    for j in range(nsup):
        if j > 0:
            h = (jnp.dot(h.astype(bf16), apow_ref[log2k - 1],
                         preferred_element_type=f32)
                 + uh_ref[:, j * k:(j + 1) * k, :].reshape(grp * k, s))
        o_ref[:, j * k:(j + 1) * k, :] = jnp.dot(
            h.astype(bf16), ct_ref[...],
            preferred_element_type=f32).astype(
                o_ref.dtype).reshape(grp, k, o_ref.shape[-1])

    hcarry_ref[...] = jnp.dot(
        h.reshape(grp, k, s)[:, k - 1, :].astype(bf16), acat_ref[...],
        preferred_element_type=f32)


def kernel(x, A, B, C):
    bsz, T, input_dim = x.shape
    state_dim = A.shape[0]
    out_dim = C.shape[0]
    out_dtype = x.dtype
    f32 = jnp.float32
    bf16 = jnp.bfloat16

    K = 32
    LOG2K = 5
    CT = 128

    G = 2 if bsz >= 16 else 1
    grp = ((-(-bsz // G) + 7) // 8) * 8
    bsz_pad = grp * G
    T_pad = -(-T // CT) * CT
    num_chunks = T_pad // CT
    nsup = CT // K
    out_pad = ((out_dim + 127) // 128) * 128

    xp = x
    if bsz_pad != bsz or T_pad != T:
        xp = jnp.pad(x, ((0, bsz_pad - bsz), (0, T_pad - T), (0, 0)))
    x_g = xp.reshape(G, grp, T_pad, input_dim)

    at = jnp.transpose(A).astype(f32)
    apows = [at]
    for _ in range(LOG2K):
        apows.append(apows[-1] @ apows[-1])
    apow_t = jnp.stack(apows[1:], axis=0).astype(bf16)
    b_t = jnp.transpose(B).astype(f32)
    btaug = jnp.concatenate([b_t, b_t @ at], axis=1).astype(bf16)
    acat = jnp.concatenate([at, at @ at], axis=1).astype(bf16)
    c_t = jnp.pad(jnp.transpose(C),
                  ((0, 0), (0, out_pad - out_dim))).astype(bf16)

    body = functools.partial(_ssm_body, grp=grp, ct=CT, k=K, log2k=LOG2K,
                             nsup=nsup)

    est_vmem = (2 * grp * CT * (input_dim * 4 + out_pad * 4)
                + grp * CT * state_dim * 4
                + 2 * (apow_t.size + btaug.size + acat.size + c_t.size) * 2)
    vmem_limit = int(min(max(2 * est_vmem, 32 << 20), 64 << 20))

    y = pl.pallas_call(
        body,
        out_shape=jax.ShapeDtypeStruct((G, grp, T_pad, out_pad), out_dtype),
        grid_spec=pltpu.PrefetchScalarGridSpec(
            num_scalar_prefetch=0,
            grid=(G, num_chunks),
            in_specs=[
                pl.BlockSpec((None, grp, CT, input_dim),
                             lambda g, c: (g, 0, c, 0)),
                pl.BlockSpec((input_dim, 2 * state_dim),
                             lambda g, c: (0, 0)),
                pl.BlockSpec((LOG2K, state_dim, state_dim),
                             lambda g, c: (0, 0, 0)),
                pl.BlockSpec((state_dim, 2 * state_dim),
                             lambda g, c: (0, 0)),
                pl.BlockSpec((state_dim, out_pad), lambda g, c: (0, 0)),
            ],
            out_specs=pl.BlockSpec((None, grp, CT, out_pad),
                                   lambda g, c: (g, 0, c, 0)),
            scratch_shapes=[
                pltpu.VMEM((grp, CT, state_dim), f32),
                pltpu.VMEM((grp, 2 * state_dim), f32),
            ],
        ),
        compiler_params=pltpu.CompilerParams(
            dimension_semantics=("parallel", "arbitrary"),
            vmem_limit_bytes=vmem_limit,
            flags={"XLA_TPU_STORE_TO_LOAD_FORWARDING_WINDOW": 12288},
        ),
    )(x_g, btaug, apow_t, acat, c_t)

    y = y.reshape(bsz_pad, T_pad, out_pad)[:bsz, :T, :out_dim]
    return y


# speedup vs baseline: 1.0376x; 1.0270x over previous
import functools

import jax
import jax.numpy as jnp
from jax.experimental import pallas as pl
from jax.experimental.pallas import tpu as pltpu


def _ssm_body(x_ref, apow_ref, bt_ref, ct_ref, o_ref, uh_ref, hcarry_ref,
              *, grp, ct, k, log2k, nsup):
    f32 = jnp.float32
    bf16 = jnp.bfloat16
    s = uh_ref.shape[-1]
    rows = grp * ct

    @pl.when(pl.program_id(1) == 0)
    def _init():
        hcarry_ref[...] = jnp.zeros_like(hcarry_ref)

    xv = x_ref[...].reshape(rows, x_ref.shape[-1]).astype(bf16)
    uh_ref[...] = jnp.dot(xv, bt_ref[...],
                          preferred_element_type=f32).reshape(grp, ct, s)

    uh_ref[:, 0:1, :] = uh_ref[:, 0:1, :] + hcarry_ref[...][:, None, :]

    for j in range(1, log2k + 1):
        m = 1 << (j - 1)
        if m % 8 == 0:
            srows = grp * (ct - m)
            w = jnp.dot(uh_ref[:, :ct - m, :].reshape(srows, s).astype(bf16),
                        apow_ref[j - 1],
                        preferred_element_type=f32).reshape(grp, ct - m, s)
            uh_ref[:, m:, :] = uh_ref[:, m:, :] + w
        else:
            w = jnp.dot(uh_ref[...].reshape(rows, s).astype(bf16),
                        apow_ref[j - 1],
                        preferred_element_type=f32).reshape(grp, ct, s)
            uh_ref[:, m:, :] = uh_ref[:, m:, :] + w[:, :ct - m, :]

    h = uh_ref[:, 0:k, :].reshape(grp * k, s)
    for j in range(nsup):
        if j > 0:
            h = (jnp.dot(h.astype(bf16), apow_ref[log2k],
                         preferred_element_type=f32)
                 + uh_ref[:, j * k:(j + 1) * k, :].reshape(grp * k, s))
        o_ref[:, j * k:(j + 1) * k, :] = jnp.dot(
            h.astype(bf16), ct_ref[...],
            preferred_element_type=f32).astype(
                o_ref.dtype).reshape(grp, k, o_ref.shape[-1])

    hcarry_ref[...] = jnp.dot(
        h.reshape(grp, k, s)[:, k - 1, :].astype(bf16), apow_ref[0],
        preferred_element_type=f32)


def kernel(x, A, B, C):
    bsz, T, input_dim = x.shape
    state_dim = A.shape[0]
    out_dim = C.shape[0]
    out_dtype = x.dtype
    f32 = jnp.float32
    bf16 = jnp.bfloat16

    K = 32
    LOG2K = 5
    CT = 128

    G = 2 if bsz >= 16 else 1
    grp = ((-(-bsz // G) + 7) // 8) * 8
    bsz_pad = grp * G
    T_pad = -(-T // CT) * CT
    num_chunks = T_pad // CT
    nsup = CT // K
    out_pad = ((out_dim + 127) // 128) * 128

    xp = x
    if bsz_pad != bsz or T_pad != T:
        xp = jnp.pad(x, ((0, bsz_pad - bsz), (0, T_pad - T), (0, 0)))
    x_g = xp.reshape(G, grp, T_pad, input_dim)

    apows = [jnp.transpose(A).astype(f32)]
    for _ in range(LOG2K):
        apows.append(apows[-1] @ apows[-1])
    apow_t = jnp.stack(apows, axis=0).astype(bf16)
    b_t = jnp.transpose(B).astype(bf16)
    c_t = jnp.pad(jnp.transpose(C),
                  ((0, 0), (0, out_pad - out_dim))).astype(bf16)

    body = functools.partial(_ssm_body, grp=grp, ct=CT, k=K, log2k=LOG2K,
                             nsup=nsup)

    est_vmem = (2 * grp * CT * (input_dim * 4 + out_pad * 4)
                + grp * CT * state_dim * 4
                + 2 * (apow_t.size + b_t.size + c_t.size) * 2)
    vmem_limit = int(min(max(2 * est_vmem, 32 << 20), 64 << 20))

    y = pl.pallas_call(
        body,
        out_shape=jax.ShapeDtypeStruct((G, grp, T_pad, out_pad), out_dtype),
        grid_spec=pltpu.PrefetchScalarGridSpec(
            num_scalar_prefetch=0,
            grid=(G, num_chunks),
            in_specs=[
                pl.BlockSpec((None, grp, CT, input_dim),
                             lambda g, c: (g, 0, c, 0)),
                pl.BlockSpec((LOG2K + 1, state_dim, state_dim),
                             lambda g, c: (0, 0, 0)),
                pl.BlockSpec((input_dim, state_dim), lambda g, c: (0, 0)),
                pl.BlockSpec((state_dim, out_pad), lambda g, c: (0, 0)),
            ],
            out_specs=pl.BlockSpec((None, grp, CT, out_pad),
                                   lambda g, c: (g, 0, c, 0)),
            scratch_shapes=[
                pltpu.VMEM((grp, CT, state_dim), f32),
                pltpu.VMEM((grp, state_dim), f32),
            ],
        ),
        compiler_params=pltpu.CompilerParams(
            dimension_semantics=("parallel", "arbitrary"),
            vmem_limit_bytes=vmem_limit,
            flags={"XLA_TPU_STORE_TO_LOAD_FORWARDING_WINDOW": 12288},
        ),
    )(x_g, apow_t, b_t, c_t)

    y = y.reshape(bsz_pad, T_pad, out_pad)[:bsz, :T, :out_dim]
    return y
